# baseline (speedup 1.0000x reference)
import functools

import jax
import jax.numpy as jnp
from jax.experimental import pallas as pl
from jax.experimental.pallas import tpu as pltpu

_Q = 20
_D = 16
_A = 1_000_000
_E = 1_000_000
_K = 1000

_GROUP = 32
_APL = 8
_ROWS = _A // _APL
_BLK = 1000
_EPAD = 1 << 20


def _stage1_body(a_ref, w_ref, o_ref):
    x = jnp.dot(a_ref[...], w_ref[...], preferred_element_type=jnp.float32)
    outs = [
        jnp.max(x[:, g * _GROUP:(g + 1) * _GROUP], axis=1, keepdims=True)
        for g in range(_APL)
    ]
    o_ref[...] = jnp.concatenate(outs, axis=1)


def _alias_scores(alias_emb, w):
    a2 = alias_emb.reshape(_ROWS, 128)
    out = pl.pallas_call(
        _stage1_body,
        grid=(_ROWS // _BLK,),
        in_specs=[
            pl.BlockSpec((_BLK, 128), lambda i: (i, 0)),
            pl.BlockSpec((128, _APL * _GROUP), lambda i: (0, 0)),
        ],
        out_specs=pl.BlockSpec((_BLK, _APL), lambda i: (i, 0)),
        out_shape=jax.ShapeDtypeStruct((_ROWS, _APL), jnp.float32),
        compiler_params=pltpu.CompilerParams(
            dimension_semantics=("arbitrary",),
        ),
    )(a2, w)
    return out.reshape(_A)


def _build_w(query):
    qpad = jnp.concatenate(
        [query, jnp.broadcast_to(query[0:1], (_GROUP - _Q, _D))], axis=0)
    w4 = jnp.einsum("ab,qd->adbq", jnp.eye(_APL, dtype=query.dtype), qpad)
    return w4.reshape(128, _APL * _GROUP)


def kernel(query, alias_emb, alias_to_entity):
    w = _build_w(query)
    alias_scores = _alias_scores(alias_emb, w)
    entity_scores = jax.ops.segment_max(
        alias_scores, alias_to_entity, num_segments=_E)
    padded = jnp.pad(entity_scores, (0, _EPAD - _E),
                     constant_values=-jnp.inf)
    topk_vals, topk_ids = jax.lax.top_k(padded, _K)
    return topk_ids, topk_vals


# speedup vs baseline: 1.0933x; 1.0933x over previous
import functools

import jax
import jax.numpy as jnp
from jax.experimental import pallas as pl
from jax.experimental.pallas import tpu as pltpu

_Q = 20
_D = 16
_A = 1_000_000
_E = 1_000_000
_K = 1000

_GROUP = 32
_APL = 8
_ROWS = _A // _APL
_BLK = 1000
_EPAD = 1 << 20


def _stage1_body(a_ref, w_ref, o_ref):
    x = jnp.dot(a_ref[...], w_ref[...], preferred_element_type=jnp.float32)
    outs = [
        jnp.max(x[:, g * _GROUP:(g + 1) * _GROUP], axis=1, keepdims=True)
        for g in range(_APL)
    ]
    o_ref[...] = jnp.concatenate(outs, axis=1)


def _alias_scores(alias_emb, w):
    a2 = alias_emb.reshape(_ROWS, 128)
    out = pl.pallas_call(
        _stage1_body,
        grid=(_ROWS // _BLK,),
        in_specs=[
            pl.BlockSpec((_BLK, 128), lambda i: (i, 0)),
            pl.BlockSpec((128, _APL * _GROUP), lambda i: (0, 0)),
        ],
        out_specs=pl.BlockSpec((_BLK, _APL), lambda i: (i, 0)),
        out_shape=jax.ShapeDtypeStruct((_ROWS, _APL), jnp.float32),
        compiler_params=pltpu.CompilerParams(
            dimension_semantics=("arbitrary",),
        ),
    )(a2, w)
    return out.reshape(_A)


def _build_w(query):
    qpad = jnp.concatenate(
        [query, jnp.broadcast_to(query[0:1], (_GROUP - _Q, _D))], axis=0)
    w4 = jnp.einsum("ab,qd->adbq", jnp.eye(_APL, dtype=query.dtype), qpad)
    return w4.reshape(128, _APL * _GROUP)


_NCH = 2048
_CW = 512
_KPAD = 1024


def _topk_body(t_ref, vals_ref, ids_ref, sc_ref, m_ref):
    sc_ref[...] = t_ref[...]
    m_ref[...] = jnp.max(t_ref[...], axis=1).reshape(16, 128)
    c_iota = (128 * jax.lax.broadcasted_iota(jnp.int32, (16, 128), 0)
              + jax.lax.broadcasted_iota(jnp.int32, (16, 128), 1))
    l_iota = jax.lax.broadcasted_iota(jnp.int32, (1, _CW), 1)
    big = jnp.int32(2**30)
    neg = jnp.float32(-jnp.inf)

    def step(k, _):
        m = m_ref[...]
        gmax = jnp.max(m)
        cid = jnp.min(jnp.where(m == gmax, c_iota, big))
        s = cid // 128
        l = cid % 128
        row = sc_ref[pl.ds(cid, 1), :]
        j = jnp.min(jnp.where(row == gmax, l_iota, big))
        vals_ref[pl.ds(k, 1), :] = gmax.reshape(1, 1)
        ids_ref[pl.ds(k, 1), :] = (_CW * cid + j).reshape(1, 1)
        row2 = jnp.where(l_iota == j, neg, row)
        sc_ref[pl.ds(cid, 1), :] = row2
        nm = jnp.max(row2)
        mrow = m_ref[pl.ds(s, 1), :]
        lrow = jax.lax.broadcasted_iota(jnp.int32, (1, 128), 1)
        m_ref[pl.ds(s, 1), :] = jnp.where(lrow == l, nm, mrow)
        return 0

    jax.lax.fori_loop(0, _K, step, 0)


def _topk(table):
    vals, ids = pl.pallas_call(
        _topk_body,
        in_specs=[pl.BlockSpec((_NCH, _CW), lambda: (0, 0))],
        out_specs=[
            pl.BlockSpec((_KPAD, 1), lambda: (0, 0)),
            pl.BlockSpec((_KPAD, 1), lambda: (0, 0)),
        ],
        out_shape=[
            jax.ShapeDtypeStruct((_KPAD, 1), jnp.float32),
            jax.ShapeDtypeStruct((_KPAD, 1), jnp.int32),
        ],
        scratch_shapes=[
            pltpu.VMEM((_NCH, _CW), jnp.float32),
            pltpu.VMEM((16, 128), jnp.float32),
        ],
    )(table)
    return vals.reshape(_KPAD)[:_K], ids.reshape(_KPAD)[:_K]


def kernel(query, alias_emb, alias_to_entity):
    w = _build_w(query)
    alias_scores = _alias_scores(alias_emb, w)
    entity_scores = jax.ops.segment_max(
        alias_scores, alias_to_entity, num_segments=_E)
    padded = jnp.pad(entity_scores, (0, _EPAD - _E),
                     constant_values=-jnp.inf)
    topk_vals, topk_ids = _topk(padded.reshape(_NCH, _CW))
    return topk_ids, topk_vals
